# baseline (speedup 1.0000x reference)
import functools

import jax
import jax.numpy as jnp
from jax import lax
from jax.experimental import pallas as pl
from jax.experimental.pallas import tpu as pltpu
from jax.experimental.pallas import tpu_sc as plsc

BATCH = 16384
NUM_FINE = 100
FINE_PAD = 112
HIST_W = 2 * FINE_PAD
NUM_SUPER_PAD = 32
NW = 32
CHUNK = BATCH // NW
L = 16
MARGIN = 0.3


def _acosh16(x):
    one = jnp.float32(1.0)
    x = jnp.maximum(x, jnp.float32(1.001))
    u = x * x - one
    ui = lax.bitcast_convert_type(u, jnp.int32)
    r = lax.bitcast_convert_type(jnp.int32(0x5F3759DF) - (ui >> 1), jnp.float32)
    half_u = jnp.float32(0.5) * u
    for _ in range(3):
        r = r * (jnp.float32(1.5) - half_u * r * r)
    t = x + u * r
    ti = lax.bitcast_convert_type(t, jnp.int32)
    e = ((ti >> 23) & jnp.int32(255)) - jnp.int32(127)
    m = lax.bitcast_convert_type(
        (ti & jnp.int32(0x007FFFFF)) | jnp.int32(0x3F800000), jnp.float32)
    big = m > jnp.float32(1.4142135)
    m = jnp.where(big, m * jnp.float32(0.5), m)
    e = jnp.where(big, e + jnp.int32(1), e)
    q = (m - one) / (m + one)
    z = q * q
    p = jnp.float32(2.0) * q * (
        one + z * (jnp.float32(1.0 / 3.0) + z * (jnp.float32(0.2) + z * (
            jnp.float32(1.0 / 7.0) + z * jnp.float32(1.0 / 9.0)))))
    return e.astype(jnp.float32) * jnp.float32(0.6931471805599453) + p


def _sc_partials(x, y):
    mesh = plsc.VectorSubcoreMesh(core_axis_name="c", subcore_axis_name="s")

    @functools.partial(
        pl.kernel,
        out_type=jax.ShapeDtypeStruct((NW, HIST_W), jnp.float32),
        mesh=mesh,
        scratch_types=[
            pltpu.VMEM((CHUNK,), jnp.float32),
            pltpu.VMEM((CHUNK,), jnp.int32),
            pltpu.VMEM((HIST_W,), jnp.float32),
            pltpu.VMEM((HIST_W,), jnp.float32),
            pltpu.VMEM((HIST_W,), jnp.float32),
            pltpu.VMEM((HIST_W,), jnp.float32),
            pltpu.SemaphoreType.DMA,
            pltpu.SemaphoreType.DMA,
        ],
        compiler_params=pltpu.CompilerParams(needs_layout_passes=False),
    )
    def body(x_hbm, y_hbm, out_hbm, x_v, y_v, hist_v, hist2_v, hist3_v, hist4_v, sem1, sem2):
        cid = lax.axis_index("c")
        sid = lax.axis_index("s")
        wid = sid * 2 + cid
        base = wid * CHUNK
        cp1 = pltpu.async_copy(x_hbm.at[pl.ds(base, CHUNK)], x_v, sem1)
        cp2 = pltpu.async_copy(y_hbm.at[pl.ds(base, CHUNK)], y_v, sem2)

        zeros = jnp.zeros((L,), jnp.float32)
        for k in range(HIST_W // L):
            hist_v[pl.ds(k * L, L)] = zeros
            hist2_v[pl.ds(k * L, L)] = zeros
            hist3_v[pl.ds(k * L, L)] = zeros
            hist4_v[pl.ds(k * L, L)] = zeros
        cp1.wait()
        cp2.wait()

        ones = jnp.ones((L,), jnp.float32)
        hists = (hist_v, hist2_v, hist3_v, hist4_v)
        for i in range(CHUNK // L):
            xv = x_v[pl.ds(i * L, L)]
            lbl = y_v[pl.ds(i * L, L)]
            d = _acosh16(xv)
            hv = hists[i % 4]
            plsc.addupdate_scatter(hv, [lbl], d)
            plsc.addupdate_scatter(hv, [lbl + jnp.int32(FINE_PAD)], ones)

        for k in range(HIST_W // L):
            hist_v[pl.ds(k * L, L)] = (
                (hist_v[pl.ds(k * L, L)] + hist2_v[pl.ds(k * L, L)])
                + (hist3_v[pl.ds(k * L, L)] + hist4_v[pl.ds(k * L, L)]))

        pltpu.sync_copy(hist_v, out_hbm.at[wid])

    return body(x, y)


def _tc_body(p_ref, lut_ref, o_ref):
    tot = jnp.sum(p_ref[...], axis=0, keepdims=True)
    fine_sum = tot[:, :NUM_FINE]
    fine_count = tot[:, FINE_PAD:FINE_PAD + NUM_FINE]
    fine_mean = fine_sum / jnp.maximum(fine_count, 1.0)
    mask_fine = (fine_count > 0).astype(jnp.float32)
    stacked = jnp.concatenate(
        [fine_sum, fine_count, fine_mean * mask_fine, mask_fine], axis=0)
    onehot_t = (lut_ref[...] == lax.broadcasted_iota(
        jnp.int32, (NUM_SUPER_PAD, NUM_FINE), 0)).astype(jnp.float32)
    seg = jax.lax.dot_general(
        stacked, onehot_t, (((1,), (1,)), ((), ())),
        preferred_element_type=jnp.float32)
    super_sum = seg[0:1]
    super_count = seg[1:2]
    fms_sum = seg[2:3]
    fcs = seg[3:4]
    super_mean = super_sum / jnp.maximum(super_count, 1.0)
    fine_mean_per_super = fms_sum / jnp.maximum(fcs, 1.0)
    mask = ((super_count > 0) & (fcs > 0)).astype(jnp.float32)
    hinge = jnp.maximum(super_mean - fine_mean_per_super + MARGIN, 0.0) ** 2
    msum = jnp.sum(mask)
    loss = jnp.where(msum > 0,
                     jnp.sum(hinge * mask) / jnp.maximum(msum, 1.0), 0.0)
    o_ref[...] = jnp.reshape(loss, (1, 1))


def kernel(cls_time, y, fine_to_super_lut):
    x = cls_time.reshape(-1)
    partials = _sc_partials(x, y)
    loss = pl.pallas_call(
        _tc_body,
        out_shape=jax.ShapeDtypeStruct((1, 1), jnp.float32),
    )(partials, fine_to_super_lut.reshape(1, NUM_FINE))
    return loss[0, 0]


# speedup vs baseline: 1.0197x; 1.0197x over previous
import functools

import jax
import jax.numpy as jnp
from jax import lax
from jax.experimental import pallas as pl
from jax.experimental.pallas import tpu as pltpu
from jax.experimental.pallas import tpu_sc as plsc

BATCH = 16384
NUM_FINE = 100
FINE_PAD = 112
HIST_W = 2 * FINE_PAD
NUM_SUPER_PAD = 32
NW = 32
CHUNK = BATCH // NW
L = 16
MARGIN = 0.3


def _acosh16(x):
    one = jnp.float32(1.0)
    x = jnp.maximum(x, jnp.float32(1.001))
    u = x * x - one
    ui = lax.bitcast_convert_type(u, jnp.int32)
    r = lax.bitcast_convert_type(jnp.int32(0x5F3759DF) - (ui >> 1), jnp.float32)
    half_u = jnp.float32(0.5) * u
    for _ in range(3):
        r = r * (jnp.float32(1.5) - half_u * r * r)
    t = x + u * r
    ti = lax.bitcast_convert_type(t, jnp.int32)
    e = ((ti >> 23) & jnp.int32(255)) - jnp.int32(127)
    m = lax.bitcast_convert_type(
        (ti & jnp.int32(0x007FFFFF)) | jnp.int32(0x3F800000), jnp.float32)
    q = (m - one) / (m + one)
    z = q * q
    p = jnp.float32(2.0) * q * (
        one + z * (jnp.float32(1.0 / 3.0) + z * (jnp.float32(0.2) + z * (
            jnp.float32(1.0 / 7.0) + z * jnp.float32(1.0 / 9.0)))))
    return e.astype(jnp.float32) * jnp.float32(0.6931471805599453) + p


def _sc_partials(x, y):
    mesh = plsc.VectorSubcoreMesh(core_axis_name="c", subcore_axis_name="s")

    @functools.partial(
        pl.kernel,
        out_type=jax.ShapeDtypeStruct((NW, HIST_W), jnp.float32),
        mesh=mesh,
        scratch_types=[
            pltpu.VMEM((CHUNK,), jnp.float32),
            pltpu.VMEM((CHUNK,), jnp.int32),
            pltpu.VMEM((HIST_W,), jnp.float32),
            pltpu.VMEM((HIST_W,), jnp.float32),
            pltpu.SemaphoreType.DMA,
            pltpu.SemaphoreType.DMA,
        ],
        compiler_params=pltpu.CompilerParams(needs_layout_passes=False),
    )
    def body(x_hbm, y_hbm, out_hbm, x_v, y_v, hist_v, hist2_v, sem1, sem2):
        cid = lax.axis_index("c")
        sid = lax.axis_index("s")
        wid = sid * 2 + cid
        base = wid * CHUNK
        cp1 = pltpu.async_copy(x_hbm.at[pl.ds(base, CHUNK)], x_v, sem1)
        cp2 = pltpu.async_copy(y_hbm.at[pl.ds(base, CHUNK)], y_v, sem2)

        zeros = jnp.zeros((L,), jnp.float32)
        for k in range(HIST_W // L):
            hist_v[pl.ds(k * L, L)] = zeros
            hist2_v[pl.ds(k * L, L)] = zeros
        cp1.wait()
        cp2.wait()

        ones = jnp.ones((L,), jnp.float32)
        for i in range(CHUNK // L):
            xv = x_v[pl.ds(i * L, L)]
            lbl = y_v[pl.ds(i * L, L)]
            d = _acosh16(xv)
            hv = hist_v if i % 2 == 0 else hist2_v
            plsc.addupdate_scatter(hv, [lbl], d)
            plsc.addupdate_scatter(hv, [lbl + jnp.int32(FINE_PAD)], ones)

        for k in range(HIST_W // L):
            hist_v[pl.ds(k * L, L)] = (
                hist_v[pl.ds(k * L, L)] + hist2_v[pl.ds(k * L, L)])

        pltpu.sync_copy(hist_v, out_hbm.at[wid])

    return body(x, y)


def _tc_body(p_ref, lut_ref, o_ref):
    tot = jnp.sum(p_ref[...], axis=0, keepdims=True)
    fine_sum = tot[:, :NUM_FINE]
    fine_count = tot[:, FINE_PAD:FINE_PAD + NUM_FINE]
    fine_mean = fine_sum / jnp.maximum(fine_count, 1.0)
    mask_fine = (fine_count > 0).astype(jnp.float32)
    stacked = jnp.concatenate(
        [fine_sum, fine_count, fine_mean * mask_fine, mask_fine], axis=0)
    onehot_t = (lut_ref[...] == lax.broadcasted_iota(
        jnp.int32, (NUM_SUPER_PAD, NUM_FINE), 0)).astype(jnp.float32)
    seg = jax.lax.dot_general(
        stacked, onehot_t, (((1,), (1,)), ((), ())),
        preferred_element_type=jnp.float32)
    super_sum = seg[0:1]
    super_count = seg[1:2]
    fms_sum = seg[2:3]
    fcs = seg[3:4]
    super_mean = super_sum / jnp.maximum(super_count, 1.0)
    fine_mean_per_super = fms_sum / jnp.maximum(fcs, 1.0)
    mask = ((super_count > 0) & (fcs > 0)).astype(jnp.float32)
    hinge = jnp.maximum(super_mean - fine_mean_per_super + MARGIN, 0.0) ** 2
    msum = jnp.sum(mask)
    loss = jnp.where(msum > 0,
                     jnp.sum(hinge * mask) / jnp.maximum(msum, 1.0), 0.0)
    o_ref[...] = jnp.reshape(loss, (1, 1))


def kernel(cls_time, y, fine_to_super_lut):
    x = cls_time.reshape(-1)
    partials = _sc_partials(x, y)
    loss = pl.pallas_call(
        _tc_body,
        out_shape=jax.ShapeDtypeStruct((1, 1), jnp.float32),
    )(partials, fine_to_super_lut.reshape(1, NUM_FINE))
    return loss[0, 0]


# speedup vs baseline: 1.1317x; 1.1098x over previous
import functools

import jax
import jax.numpy as jnp
from jax import lax
from jax.experimental import pallas as pl
from jax.experimental.pallas import tpu as pltpu
from jax.experimental.pallas import tpu_sc as plsc

BATCH = 16384
NUM_FINE = 100
FINE_PAD = 112
HIST_W = 2 * FINE_PAD
NUM_SUPER_PAD = 32
NW = 32
CHUNK = BATCH // NW
L = 16
MARGIN = 0.3


def _acosh16(x):
    one = jnp.float32(1.0)
    x = jnp.maximum(x, jnp.float32(1.001))
    u = x * x - one
    ui = lax.bitcast_convert_type(u, jnp.int32)
    r = lax.bitcast_convert_type(jnp.int32(0x5F3759DF) - (ui >> 1), jnp.float32)
    half_u = jnp.float32(0.5) * u
    for _ in range(3):
        r = r * (jnp.float32(1.5) - half_u * r * r)
    t = x + u * r
    ti = lax.bitcast_convert_type(t, jnp.int32)
    e = ((ti >> 23) & jnp.int32(255)) - jnp.int32(127)
    m = lax.bitcast_convert_type(
        (ti & jnp.int32(0x007FFFFF)) | jnp.int32(0x3F800000), jnp.float32)
    q = (m - one) / (m + one)
    z = q * q
    p = jnp.float32(2.0) * q * (
        one + z * (jnp.float32(1.0 / 3.0) + z * (jnp.float32(0.2) + z * (
            jnp.float32(1.0 / 7.0) + z * jnp.float32(1.0 / 9.0)))))
    return e.astype(jnp.float32) * jnp.float32(0.6931471805599453) + p


def _sc_partials(x, y):
    mesh = plsc.VectorSubcoreMesh(core_axis_name="c", subcore_axis_name="s")

    @functools.partial(
        pl.kernel,
        out_type=jax.ShapeDtypeStruct((NW, HIST_W), jnp.float32),
        mesh=mesh,
        scratch_types=[
            pltpu.VMEM((CHUNK,), jnp.float32),
            pltpu.VMEM((CHUNK,), jnp.int32),
            pltpu.VMEM((HIST_W,), jnp.float32),
            pltpu.VMEM((HIST_W,), jnp.float32),
            pltpu.SemaphoreType.DMA,
            pltpu.SemaphoreType.DMA,
        ],
        compiler_params=pltpu.CompilerParams(needs_layout_passes=False),
    )
    def body(x_hbm, y_hbm, out_hbm, x_v, y_v, hist_v, hist2_v, sem1, sem2):
        cid = lax.axis_index("c")
        sid = lax.axis_index("s")
        wid = sid * 2 + cid
        base = wid * CHUNK
        cp1 = pltpu.async_copy(x_hbm.at[pl.ds(base, CHUNK)], x_v, sem1)
        cp2 = pltpu.async_copy(y_hbm.at[pl.ds(base, CHUNK)], y_v, sem2)

        zeros = jnp.zeros((L,), jnp.float32)

        def zero_body(k, _):
            hist_v[pl.ds(k * L, L)] = zeros
            hist2_v[pl.ds(k * L, L)] = zeros
            return _

        lax.fori_loop(0, HIST_W // L, zero_body, None)
        cp1.wait()
        cp2.wait()

        ones = jnp.ones((L,), jnp.float32)

        def main_body(i, _):
            b = i * (2 * L)
            xv = x_v[pl.ds(b, L)]
            lbl = y_v[pl.ds(b, L)]
            d = _acosh16(xv)
            plsc.addupdate_scatter(hist_v, [lbl], d)
            plsc.addupdate_scatter(hist_v, [lbl + jnp.int32(FINE_PAD)], ones)
            xv2 = x_v[pl.ds(b + L, L)]
            lbl2 = y_v[pl.ds(b + L, L)]
            d2 = _acosh16(xv2)
            plsc.addupdate_scatter(hist2_v, [lbl2], d2)
            plsc.addupdate_scatter(hist2_v, [lbl2 + jnp.int32(FINE_PAD)], ones)
            return _

        lax.fori_loop(0, CHUNK // (2 * L), main_body, None)

        def merge_body(k, _):
            hist_v[pl.ds(k * L, L)] = (
                hist_v[pl.ds(k * L, L)] + hist2_v[pl.ds(k * L, L)])
            return _

        lax.fori_loop(0, HIST_W // L, merge_body, None)

        pltpu.sync_copy(hist_v, out_hbm.at[wid])

    return body(x, y)


def _tc_body(p_ref, lut_ref, o_ref):
    tot = jnp.sum(p_ref[...], axis=0, keepdims=True)
    fine_sum = tot[:, :NUM_FINE]
    fine_count = tot[:, FINE_PAD:FINE_PAD + NUM_FINE]
    fine_mean = fine_sum / jnp.maximum(fine_count, 1.0)
    mask_fine = (fine_count > 0).astype(jnp.float32)
    stacked = jnp.concatenate(
        [fine_sum, fine_count, fine_mean * mask_fine, mask_fine], axis=0)
    onehot_t = (lut_ref[...] == lax.broadcasted_iota(
        jnp.int32, (NUM_SUPER_PAD, NUM_FINE), 0)).astype(jnp.float32)
    seg = jax.lax.dot_general(
        stacked, onehot_t, (((1,), (1,)), ((), ())),
        preferred_element_type=jnp.float32)
    super_sum = seg[0:1]
    super_count = seg[1:2]
    fms_sum = seg[2:3]
    fcs = seg[3:4]
    super_mean = super_sum / jnp.maximum(super_count, 1.0)
    fine_mean_per_super = fms_sum / jnp.maximum(fcs, 1.0)
    mask = ((super_count > 0) & (fcs > 0)).astype(jnp.float32)
    hinge = jnp.maximum(super_mean - fine_mean_per_super + MARGIN, 0.0) ** 2
    msum = jnp.sum(mask)
    loss = jnp.where(msum > 0,
                     jnp.sum(hinge * mask) / jnp.maximum(msum, 1.0), 0.0)
    o_ref[...] = jnp.reshape(loss, (1, 1))


def kernel(cls_time, y, fine_to_super_lut):
    x = cls_time.reshape(-1)
    partials = _sc_partials(x, y)
    loss = pl.pallas_call(
        _tc_body,
        out_shape=jax.ShapeDtypeStruct((1, 1), jnp.float32),
    )(partials, fine_to_super_lut.reshape(1, NUM_FINE))
    return loss[0, 0]


# speedup vs baseline: 1.1489x; 1.0152x over previous
import functools

import jax
import jax.numpy as jnp
from jax import lax
from jax.experimental import pallas as pl
from jax.experimental.pallas import tpu as pltpu
from jax.experimental.pallas import tpu_sc as plsc

BATCH = 16384
NUM_FINE = 100
FINE_PAD = 112
HIST_W = 2 * FINE_PAD
NUM_SUPER_PAD = 32
NW = 32
CHUNK = BATCH // NW
L = 16
MARGIN = 0.3


def _acosh16(x):
    one = jnp.float32(1.0)
    x = jnp.maximum(x, jnp.float32(1.001))
    u = x * x - one
    ui = lax.bitcast_convert_type(u, jnp.int32)
    r = lax.bitcast_convert_type(jnp.int32(0x5F3759DF) - (ui >> 1), jnp.float32)
    half_u = jnp.float32(0.5) * u
    for _ in range(3):
        r = r * (jnp.float32(1.5) - half_u * r * r)
    t = x + u * r
    ti = lax.bitcast_convert_type(t, jnp.int32)
    e = ((ti >> 23) & jnp.int32(255)) - jnp.int32(127)
    m = lax.bitcast_convert_type(
        (ti & jnp.int32(0x007FFFFF)) | jnp.int32(0x3F800000), jnp.float32)
    q = (m - one) / (m + one)
    z = q * q
    p = jnp.float32(2.0) * q * (
        one + z * (jnp.float32(1.0 / 3.0) + z * (jnp.float32(0.2) + z * (
            jnp.float32(1.0 / 7.0) + z * jnp.float32(1.0 / 9.0)))))
    return e.astype(jnp.float32) * jnp.float32(0.6931471805599453) + p


def _sc_partials(x, y):
    mesh = plsc.VectorSubcoreMesh(core_axis_name="c", subcore_axis_name="s")

    @functools.partial(
        pl.kernel,
        out_type=jax.ShapeDtypeStruct((NW, HIST_W), jnp.float32),
        mesh=mesh,
        scratch_types=[
            pltpu.VMEM((CHUNK,), jnp.float32),
            pltpu.VMEM((CHUNK,), jnp.int32),
            pltpu.VMEM((HIST_W,), jnp.float32),
            pltpu.VMEM((HIST_W,), jnp.float32),
            pltpu.SemaphoreType.DMA,
            pltpu.SemaphoreType.DMA,
        ],
        compiler_params=pltpu.CompilerParams(needs_layout_passes=False),
    )
    def body(x_hbm, y_hbm, out_hbm, x_v, y_v, hist_v, hist2_v, sem1, sem2):
        cid = lax.axis_index("c")
        sid = lax.axis_index("s")
        wid = sid * 2 + cid
        base = wid * CHUNK
        cp1 = pltpu.async_copy(x_hbm.at[pl.ds(base, CHUNK)], x_v, sem1)
        cp2 = pltpu.async_copy(y_hbm.at[pl.ds(base, CHUNK)], y_v, sem2)

        zeros = jnp.zeros((L,), jnp.float32)

        def zero_body(k, _):
            hist_v[pl.ds(k * L, L)] = zeros
            return _

        lax.fori_loop(0, HIST_W // L, zero_body, None)
        cp1.wait()
        cp2.wait()

        ones = jnp.ones((L,), jnp.float32)

        def main_body(i, _):
            b = i * L
            xv = x_v[pl.ds(b, L)]
            lbl = y_v[pl.ds(b, L)]
            d = _acosh16(xv)
            plsc.addupdate_scatter(hist_v, [lbl], d)
            plsc.addupdate_scatter(hist_v, [lbl + jnp.int32(FINE_PAD)], ones)
            return _

        lax.fori_loop(0, CHUNK // L, main_body, None)

        pltpu.sync_copy(hist_v, out_hbm.at[wid])

    return body(x, y)


def _tc_body(p_ref, lut_ref, o_ref):
    tot = jnp.sum(p_ref[...], axis=0, keepdims=True)
    fine_sum = tot[:, :NUM_FINE]
    fine_count = tot[:, FINE_PAD:FINE_PAD + NUM_FINE]
    fine_mean = fine_sum / jnp.maximum(fine_count, 1.0)
    mask_fine = (fine_count > 0).astype(jnp.float32)
    stacked = jnp.concatenate(
        [fine_sum, fine_count, fine_mean * mask_fine, mask_fine], axis=0)
    onehot_t = (lut_ref[...] == lax.broadcasted_iota(
        jnp.int32, (NUM_SUPER_PAD, NUM_FINE), 0)).astype(jnp.float32)
    seg = jax.lax.dot_general(
        stacked, onehot_t, (((1,), (1,)), ((), ())),
        preferred_element_type=jnp.float32)
    super_sum = seg[0:1]
    super_count = seg[1:2]
    fms_sum = seg[2:3]
    fcs = seg[3:4]
    super_mean = super_sum / jnp.maximum(super_count, 1.0)
    fine_mean_per_super = fms_sum / jnp.maximum(fcs, 1.0)
    mask = ((super_count > 0) & (fcs > 0)).astype(jnp.float32)
    hinge = jnp.maximum(super_mean - fine_mean_per_super + MARGIN, 0.0) ** 2
    msum = jnp.sum(mask)
    loss = jnp.where(msum > 0,
                     jnp.sum(hinge * mask) / jnp.maximum(msum, 1.0), 0.0)
    o_ref[...] = jnp.reshape(loss, (1, 1))


def kernel(cls_time, y, fine_to_super_lut):
    x = cls_time.reshape(-1)
    partials = _sc_partials(x, y)
    loss = pl.pallas_call(
        _tc_body,
        out_shape=jax.ShapeDtypeStruct((1, 1), jnp.float32),
    )(partials, fine_to_super_lut.reshape(1, NUM_FINE))
    return loss[0, 0]
